# baseline (speedup 1.0000x reference)
import functools

import numpy as np
import jax
import jax.numpy as jnp
from jax import lax
from jax.experimental import pallas as pl
from jax.experimental.pallas import tpu as pltpu
from jax.experimental.pallas import tpu_sc as plsc

SEG = 16
NSEG = 256

_PERM = np.asarray([
    121, 35, 130, 148, 197, 45, 176, 179, 139, 188, 99, 144, 152, 189, 31,
    112, 85, 63, 117, 174, 114, 254, 82, 65, 7, 4, 101, 102, 78, 163, 157,
    183, 29, 240, 177, 108, 83, 129, 212, 44, 211, 16, 58, 123, 37, 111, 19,
    61, 2, 142, 34, 156, 5, 90, 175, 167, 251, 110, 72, 155, 178, 219, 153,
    30, 42, 186, 246, 3, 70, 67, 223, 39, 56, 192, 169, 218, 195, 173, 245,
    241, 69, 80, 22, 6, 199, 118, 235, 54, 77, 147, 18, 249, 10, 11, 234, 53,
    236, 94, 32, 217, 159, 15, 184, 49, 137, 50, 138, 20, 237, 253, 185, 43,
    92, 8, 140, 233, 24, 81, 239, 96, 154, 135, 160, 106, 128, 191, 9, 200,
    40, 187, 71, 248, 164, 207, 93, 59, 201, 158, 210, 75, 131, 97, 66, 25,
    196, 242, 206, 243, 238, 73, 13, 52, 203, 202, 255, 194, 88, 250, 62,
    230, 150, 209, 132, 87, 76, 198, 60, 244, 47, 33, 79, 180, 247, 14, 228,
    17, 38, 86, 231, 190, 232, 23, 105, 220, 0, 145, 213, 226, 133, 41, 64,
    21, 161, 166, 124, 116, 26, 165, 168, 193, 57, 208, 181, 89, 146, 182,
    126, 125, 1, 115, 28, 113, 225, 172, 162, 48, 170, 227, 36, 252, 119,
    151, 120, 224, 122, 100, 91, 222, 55, 103, 51, 215, 127, 98, 107, 27, 74,
    136, 229, 204, 221, 12, 134, 109, 84, 205, 171, 143, 68, 216, 149, 141,
    104, 95, 214, 46,
], dtype=np.int32)


def _perm_cycles(perm: np.ndarray):
    seen = np.zeros(len(perm), dtype=bool)
    cycles = []
    for start in range(len(perm)):
        if seen[start]:
            continue
        cyc = [start]
        seen[start] = True
        j = int(perm[start])
        while j != start:
            cyc.append(j)
            seen[j] = True
            j = int(perm[j])
        if len(cyc) > 1:
            cycles.append(cyc)
    return cycles


_CYCLES = _perm_cycles(_PERM)

NC = 2
NS = 16
NW = NC * NS

TROW = 8
NBUF = 3


@functools.cache
def _make_permute(n_rows: int, n_cols: int):
    n_trows = n_rows // TROW
    nch = n_trows // NW
    mesh = plsc.VectorSubcoreMesh(core_axis_name="c", subcore_axis_name="s")

    @functools.partial(
        pl.kernel,
        out_type=jax.ShapeDtypeStruct((n_rows, n_cols), jnp.float32),
        mesh=mesh,
        scratch_types=[
            pltpu.VMEM((NBUF * TROW, n_cols), jnp.float32),
            pltpu.SemaphoreType.DMA((NBUF,)),
            pltpu.SemaphoreType.DMA((NBUF,)),
        ],
    )
    def permute_kernel(x_hbm, out_hbm, bufs, isem, osem):
        wid = lax.axis_index("s") * NC + lax.axis_index("c")
        base = pl.multiple_of(wid * (nch * TROW), TROW)

        def bufref(b):
            return bufs.at[pl.ds(pl.multiple_of(b * TROW, TROW), TROW)]

        def fire_in(c, b):
            row0 = pl.multiple_of(base + c * TROW, TROW)
            pltpu.async_copy(x_hbm.at[pl.ds(row0, TROW)], bufref(b),
                             isem.at[b])

        def wait_in(c, b):
            row0 = pl.multiple_of(base + c * TROW, TROW)
            pltpu.make_async_copy(
                x_hbm.at[pl.ds(row0, TROW)], bufref(b), isem.at[b]
            ).wait()

        def fire_out(c, b):
            row0 = pl.multiple_of(base + c * TROW, TROW)
            pltpu.async_copy(bufref(b), out_hbm.at[pl.ds(row0, TROW)],
                             osem.at[b])

        def wait_out(c, b):
            row0 = pl.multiple_of(base + c * TROW, TROW)
            pltpu.make_async_copy(
                bufref(b), out_hbm.at[pl.ds(row0, TROW)], osem.at[b]
            ).wait()

        def perm_chunk(b):
            buf = bufs
            roff = b * TROW

            def body(sl0, carry):
                sl = roff + sl0
                for cyc in _CYCLES:
                    tmp = buf[sl, pl.ds(SEG * cyc[0], SEG)]
                    for dst, src in zip(cyc[:-1], cyc[1:]):
                        buf[sl, pl.ds(SEG * dst, SEG)] = (
                            buf[sl, pl.ds(SEG * src, SEG)]
                        )
                    buf[sl, pl.ds(SEG * cyc[-1], SEG)] = tmp
                return carry

            lax.fori_loop(0, TROW, body, 0)

        fire_in(0, 0)
        fire_in(1, 1)

        def slot(c, carry):
            b = lax.rem(c, NBUF)
            bn = lax.rem(c + 2, NBUF)

            @pl.when(c >= 1)
            def _():
                wait_out(c - 1, bn)

            @pl.when(c + 2 < nch)
            def _():
                fire_in(c + 2, bn)

            wait_in(c, b)
            perm_chunk(b)
            fire_out(c, b)
            return carry

        lax.fori_loop(0, nch, slot, 0)
        wait_out(nch - 1, (nch - 1) % NBUF)

    return permute_kernel


def kernel(x):
    lead = x.shape[:-1]
    time_steps = x.shape[-1]
    num_segments = time_steps // SEG
    if num_segments <= 1:
        return x
    n_rows = int(np.prod(lead))
    x2 = x.reshape(n_rows, time_steps)
    out = _make_permute(n_rows, time_steps)(x2)
    return out.reshape(*lead, time_steps)


# speedup vs baseline: 1.0137x; 1.0137x over previous
import functools

import numpy as np
import jax
import jax.numpy as jnp
from jax import lax
from jax.experimental import pallas as pl
from jax.experimental.pallas import tpu as pltpu
from jax.experimental.pallas import tpu_sc as plsc

SEG = 16
NSEG = 256

_PERM = np.asarray([
    121, 35, 130, 148, 197, 45, 176, 179, 139, 188, 99, 144, 152, 189, 31,
    112, 85, 63, 117, 174, 114, 254, 82, 65, 7, 4, 101, 102, 78, 163, 157,
    183, 29, 240, 177, 108, 83, 129, 212, 44, 211, 16, 58, 123, 37, 111, 19,
    61, 2, 142, 34, 156, 5, 90, 175, 167, 251, 110, 72, 155, 178, 219, 153,
    30, 42, 186, 246, 3, 70, 67, 223, 39, 56, 192, 169, 218, 195, 173, 245,
    241, 69, 80, 22, 6, 199, 118, 235, 54, 77, 147, 18, 249, 10, 11, 234, 53,
    236, 94, 32, 217, 159, 15, 184, 49, 137, 50, 138, 20, 237, 253, 185, 43,
    92, 8, 140, 233, 24, 81, 239, 96, 154, 135, 160, 106, 128, 191, 9, 200,
    40, 187, 71, 248, 164, 207, 93, 59, 201, 158, 210, 75, 131, 97, 66, 25,
    196, 242, 206, 243, 238, 73, 13, 52, 203, 202, 255, 194, 88, 250, 62,
    230, 150, 209, 132, 87, 76, 198, 60, 244, 47, 33, 79, 180, 247, 14, 228,
    17, 38, 86, 231, 190, 232, 23, 105, 220, 0, 145, 213, 226, 133, 41, 64,
    21, 161, 166, 124, 116, 26, 165, 168, 193, 57, 208, 181, 89, 146, 182,
    126, 125, 1, 115, 28, 113, 225, 172, 162, 48, 170, 227, 36, 252, 119,
    151, 120, 224, 122, 100, 91, 222, 55, 103, 51, 215, 127, 98, 107, 27, 74,
    136, 229, 204, 221, 12, 134, 109, 84, 205, 171, 143, 68, 216, 149, 141,
    104, 95, 214, 46,
], dtype=np.int32)


def _perm_cycles(perm: np.ndarray):
    seen = np.zeros(len(perm), dtype=bool)
    cycles = []
    for start in range(len(perm)):
        if seen[start]:
            continue
        cyc = [start]
        seen[start] = True
        j = int(perm[start])
        while j != start:
            cyc.append(j)
            seen[j] = True
            j = int(perm[j])
        if len(cyc) > 1:
            cycles.append(cyc)
    return cycles


_CYCLES = _perm_cycles(_PERM)

NC = 2
NS = 16
NW = NC * NS

TROW = 8
NBUF = 3


@functools.cache
def _make_permute(n_rows: int, n_cols: int):
    n_trows = n_rows // TROW
    nch = n_trows // NW
    mesh = plsc.VectorSubcoreMesh(core_axis_name="c", subcore_axis_name="s")

    @functools.partial(
        pl.kernel,
        out_type=jax.ShapeDtypeStruct((n_rows, n_cols), jnp.float32),
        mesh=mesh,
        scratch_types=[
            [pltpu.VMEM((TROW, n_cols), jnp.float32) for _ in range(NBUF)],
            [pltpu.SemaphoreType.DMA for _ in range(NBUF)],
            [pltpu.SemaphoreType.DMA for _ in range(NBUF)],
        ],
    )
    def permute_kernel(x_hbm, out_hbm, bufs, isem, osem):
        wid = lax.axis_index("s") * NC + lax.axis_index("c")
        base = pl.multiple_of(wid * (nch * TROW), TROW)

        def fire_in(c, b):
            row0 = pl.multiple_of(base + c * TROW, TROW)
            pltpu.async_copy(x_hbm.at[pl.ds(row0, TROW)], bufs[b], isem[b])

        def wait_in(c, b):
            row0 = pl.multiple_of(base + c * TROW, TROW)
            pltpu.make_async_copy(
                x_hbm.at[pl.ds(row0, TROW)], bufs[b], isem[b]
            ).wait()

        def fire_out(c, b):
            row0 = pl.multiple_of(base + c * TROW, TROW)
            pltpu.async_copy(bufs[b], out_hbm.at[pl.ds(row0, TROW)], osem[b])

        def wait_out(c, b):
            row0 = pl.multiple_of(base + c * TROW, TROW)
            pltpu.make_async_copy(
                bufs[b], out_hbm.at[pl.ds(row0, TROW)], osem[b]
            ).wait()

        def perm_chunk(b):
            buf = bufs[b]

            def body(sl, carry):
                for cyc in _CYCLES:
                    tmp = buf[sl, pl.ds(SEG * cyc[0], SEG)]
                    for dst, src in zip(cyc[:-1], cyc[1:]):
                        buf[sl, pl.ds(SEG * dst, SEG)] = (
                            buf[sl, pl.ds(SEG * src, SEG)]
                        )
                    buf[sl, pl.ds(SEG * cyc[-1], SEG)] = tmp
                return carry

            lax.fori_loop(0, TROW, body, 0)

        fire_in(0, 0)
        fire_in(1, 1)

        n_groups = (nch + 2) // 3

        def group(g, carry):
            for k in range(3):
                c = 3 * g + k
                b = k
                bn = (b + 2) % NBUF

                @pl.when((c + 2 < nch) & (c >= 1))
                def _():
                    wait_out(c - 1, bn)

                @pl.when(c + 2 < nch)
                def _():
                    fire_in(c + 2, bn)

                @pl.when(c < nch)
                def _():
                    wait_in(c, b)
                    perm_chunk(b)
                    fire_out(c, b)

            return carry

        lax.fori_loop(0, n_groups, group, 0)

        for c in range(nch - NBUF, nch):
            wait_out(c, c % NBUF)

    return permute_kernel


def kernel(x):
    lead = x.shape[:-1]
    time_steps = x.shape[-1]
    num_segments = time_steps // SEG
    if num_segments <= 1:
        return x
    n_rows = int(np.prod(lead))
    x2 = x.reshape(n_rows, time_steps)
    out = _make_permute(n_rows, time_steps)(x2)
    return out.reshape(*lead, time_steps)
